# Initial kernel scaffold; baseline (speedup 1.0000x reference)
import functools

import jax
import jax.numpy as jnp
from jax import lax
from jax.experimental import pallas as pl
from jax.experimental.pallas import tpu as pltpu
from jax.experimental.pallas import tpu_sc as plsc

N = 10000
E = 320000
D = 128
DO = 64

NC = 2
NS = 16
NW = NC * NS

NPAD = 10240
ROWS_PER_TILE = NPAD // NS
EB = 80
EPW = E // NW
NBLK = EPW // EB
DEGW = 16

_MESH = plsc.VectorSubcoreMesh(
    core_axis_name="c", subcore_axis_name="s", num_cores=NC, num_subcores=NS)


@functools.partial(
    pl.kernel,
    mesh=_MESH,
    out_type=jax.ShapeDtypeStruct((NC, NPAD, DEGW), jnp.float32),
    scratch_types=[
        pltpu.VMEM((EB,), jnp.int32),
        pltpu.VMEM((EB, DEGW), jnp.float32),
        pltpu.VMEM_SHARED((NPAD, DEGW), jnp.float32),
    ],
)
def _sc_degree(dst_hbm, out_hbm, idx_v, ones_v, acc_sh):
    cid = lax.axis_index("c")
    sid = lax.axis_index("s")
    wid = sid * NC + cid

    zero16 = jnp.zeros((16,), jnp.float32)
    for i in range(EB):
        ones_v[i, :] = zero16
    for k in range(ROWS_PER_TILE // EB):
        pltpu.sync_copy(ones_v, acc_sh.at[pl.ds(sid * ROWS_PER_TILE + k * EB, EB)])
    one16 = jnp.ones((16,), jnp.float32)
    for i in range(EB):
        ones_v[i, :] = one16
    plsc.subcore_barrier()

    base = wid * EPW

    def body(i, carry):
        off = pl.multiple_of(base + i * EB, 8)
        pltpu.sync_copy(dst_hbm.at[pl.ds(off, EB)], idx_v)
        pltpu.sync_copy(ones_v, acc_sh.at[idx_v], add=True)
        return carry

    lax.fori_loop(0, NBLK, body, 0)
    plsc.subcore_barrier()
    pltpu.sync_copy(acc_sh.at[pl.ds(sid * ROWS_PER_TILE, ROWS_PER_TILE)],
                    out_hbm.at[cid, pl.ds(sid * ROWS_PER_TILE, ROWS_PER_TILE)])


@functools.partial(
    pl.kernel,
    mesh=_MESH,
    out_type=jax.ShapeDtypeStruct((NC, NPAD, D), jnp.float32),
    scratch_types=[
        pltpu.VMEM((EB,), jnp.int32),
        pltpu.VMEM((EB,), jnp.int32),
        pltpu.VMEM((EB, D), jnp.float32),
        pltpu.VMEM_SHARED((NPAD, D), jnp.float32),
        pltpu.SemaphoreType.DMA,
    ],
)
def _sc_prop(f_hbm, src_hbm, dst_hbm, out_hbm, isrc_v, idst_v, rows_v, acc_sh, sem):
    cid = lax.axis_index("c")
    sid = lax.axis_index("s")
    wid = sid * NC + cid

    zero16 = jnp.zeros((16,), jnp.float32)
    for i in range(EB):
        for j in range(D // 16):
            rows_v[i, pl.ds(j * 16, 16)] = zero16
    for k in range(ROWS_PER_TILE // EB):
        pltpu.sync_copy(rows_v, acc_sh.at[pl.ds(sid * ROWS_PER_TILE + k * EB, EB)])
    plsc.subcore_barrier()

    base = wid * EPW

    def body(i, carry):
        off = pl.multiple_of(base + i * EB, 8)
        pltpu.sync_copy(src_hbm.at[pl.ds(off, EB)], isrc_v)
        pltpu.sync_copy(dst_hbm.at[pl.ds(off, EB)], idst_v)
        pltpu.async_copy(f_hbm.at[isrc_v], rows_v, sem).wait()
        pltpu.sync_copy(rows_v, acc_sh.at[idst_v], add=True)
        return carry

    lax.fori_loop(0, NBLK, body, 0)
    plsc.subcore_barrier()
    pltpu.sync_copy(acc_sh.at[pl.ds(sid * ROWS_PER_TILE, ROWS_PER_TILE)],
                    out_hbm.at[cid, pl.ds(sid * ROWS_PER_TILE, ROWS_PER_TILE)])


R = 512
GRID = (NPAD // R,)


def _dinv_block(degp_ref):
    d = degp_ref[0, :, 0:1] + degp_ref[1, :, 0:1] + 1.0
    return lax.rsqrt(d)


def _t1_body(degp_ref, x_ref, xp_ref):
    xp_ref[...] = _dinv_block(degp_ref) * x_ref[...]


_t1 = pl.pallas_call(
    _t1_body,
    grid=GRID,
    in_specs=[
        pl.BlockSpec((NC, R, DEGW), lambda i: (0, i, 0)),
        pl.BlockSpec((R, D), lambda i: (i, 0)),
    ],
    out_specs=pl.BlockSpec((R, D), lambda i: (i, 0)),
    out_shape=jax.ShapeDtypeStruct((N, D), jnp.float32),
)


def _t2_body(acc_ref, xp_ref, degp_ref, w_ref, b_ref, hp_ref):
    dinv = _dinv_block(degp_ref)
    s = dinv * (acc_ref[0] + acc_ref[1] + xp_ref[...])
    h = jnp.dot(s, w_ref[...], preferred_element_type=jnp.float32) + b_ref[...]
    hp_ref[...] = dinv * jnp.maximum(h, 0.0)


_t2 = pl.pallas_call(
    _t2_body,
    grid=GRID,
    in_specs=[
        pl.BlockSpec((NC, R, D), lambda i: (0, i, 0)),
        pl.BlockSpec((R, D), lambda i: (i, 0)),
        pl.BlockSpec((NC, R, DEGW), lambda i: (0, i, 0)),
        pl.BlockSpec((D, D), lambda i: (0, 0)),
        pl.BlockSpec((1, D), lambda i: (0, 0)),
    ],
    out_specs=pl.BlockSpec((R, D), lambda i: (i, 0)),
    out_shape=jax.ShapeDtypeStruct((N, D), jnp.float32),
)


def _t3_body(acc_ref, hp_ref, degp_ref, wmu_ref, bmu_ref, wls_ref, bls_ref,
             mu_ref, ls_ref):
    dinv = _dinv_block(degp_ref)
    g = dinv * (acc_ref[0] + acc_ref[1] + hp_ref[...])
    mu_ref[...] = jnp.dot(g, wmu_ref[...], preferred_element_type=jnp.float32) + bmu_ref[...]
    ls_ref[...] = jnp.dot(g, wls_ref[...], preferred_element_type=jnp.float32) + bls_ref[...]


_t3 = pl.pallas_call(
    _t3_body,
    grid=GRID,
    in_specs=[
        pl.BlockSpec((NC, R, D), lambda i: (0, i, 0)),
        pl.BlockSpec((R, D), lambda i: (i, 0)),
        pl.BlockSpec((NC, R, DEGW), lambda i: (0, i, 0)),
        pl.BlockSpec((D, DO), lambda i: (0, 0)),
        pl.BlockSpec((1, DO), lambda i: (0, 0)),
        pl.BlockSpec((D, DO), lambda i: (0, 0)),
        pl.BlockSpec((1, DO), lambda i: (0, 0)),
    ],
    out_specs=[
        pl.BlockSpec((R, DO), lambda i: (i, 0)),
        pl.BlockSpec((R, DO), lambda i: (i, 0)),
    ],
    out_shape=[
        jax.ShapeDtypeStruct((N, DO), jnp.float32),
        jax.ShapeDtypeStruct((N, DO), jnp.float32),
    ],
)


def kernel(x, edge_index, W1, b1, Wmu, bmu, Wls, bls):
    src = edge_index[0]
    dst = edge_index[1]
    degp = _sc_degree(dst)
    xp = _t1(degp, x)
    acc1 = _sc_prop(xp, src, dst)
    hp = _t2(acc1, xp, degp, W1, b1.reshape(1, D))
    acc2 = _sc_prop(hp, src, dst)
    mu, logstd = _t3(acc2, hp, degp, Wmu, bmu.reshape(1, DO),
                     Wls, bls.reshape(1, DO))
    return (mu, logstd)


# speedup vs baseline: 15.3941x; 15.3941x over previous
import functools

import jax
import jax.numpy as jnp
from jax import lax
from jax.experimental import pallas as pl
from jax.experimental.pallas import tpu as pltpu
from jax.experimental.pallas import tpu_sc as plsc

N = 10000
E = 320000
D = 128
DO = 64

NC = 2
NS = 16
NW = NC * NS

NPAD = 10240
ROWS_PER_TILE = NPAD // NS
EB = 80
EPW = E // NW
NBLK = EPW // EB
DEGW = 16

def _mesh():
    return plsc.VectorSubcoreMesh(
        core_axis_name="c", subcore_axis_name="s", num_cores=NC, num_subcores=NS)


def _sc_degree_body(dst_hbm, out_hbm, idx_v, ones_v, acc_sh):
    cid = lax.axis_index("c")
    sid = lax.axis_index("s")
    wid = sid * NC + cid

    zero16 = jnp.zeros((16,), jnp.float32)
    for i in range(EB):
        ones_v[i, :] = zero16
    for k in range(ROWS_PER_TILE // EB):
        pltpu.sync_copy(ones_v, acc_sh.at[pl.ds(sid * ROWS_PER_TILE + k * EB, EB)])
    one16 = jnp.ones((16,), jnp.float32)
    for i in range(EB):
        ones_v[i, :] = one16
    plsc.subcore_barrier()

    base = wid * EPW

    def body(i, carry):
        off = pl.multiple_of(base + i * EB, 8)
        pltpu.sync_copy(dst_hbm.at[pl.ds(off, EB)], idx_v)
        pltpu.sync_copy(ones_v, acc_sh.at[idx_v], add=True)
        return carry

    lax.fori_loop(0, NBLK, body, 0)
    plsc.subcore_barrier()
    pltpu.sync_copy(acc_sh.at[pl.ds(sid * ROWS_PER_TILE, ROWS_PER_TILE)],
                    out_hbm.at[cid, pl.ds(sid * ROWS_PER_TILE, ROWS_PER_TILE)])


@functools.cache
def _sc_degree():
    return pl.kernel(
        _sc_degree_body,
        mesh=_mesh(),
        out_type=jax.ShapeDtypeStruct((NC, NPAD, DEGW), jnp.float32),
        scratch_types=[
            pltpu.VMEM((EB,), jnp.int32),
            pltpu.VMEM((EB, DEGW), jnp.float32),
            pltpu.VMEM_SHARED((NPAD, DEGW), jnp.float32),
        ],
    )


def _sc_prop_body(f_hbm, src_hbm, dst_hbm, out_hbm, isrc_v, idst_v, rows_v, acc_sh, sem):
    cid = lax.axis_index("c")
    sid = lax.axis_index("s")
    wid = sid * NC + cid

    zero16 = jnp.zeros((16,), jnp.float32)
    for i in range(EB):
        for j in range(D // 16):
            rows_v[i, pl.ds(j * 16, 16)] = zero16
    for k in range(ROWS_PER_TILE // EB):
        pltpu.sync_copy(rows_v, acc_sh.at[pl.ds(sid * ROWS_PER_TILE + k * EB, EB)])
    plsc.subcore_barrier()

    base = wid * EPW

    def body(i, carry):
        off = pl.multiple_of(base + i * EB, 8)
        pltpu.sync_copy(src_hbm.at[pl.ds(off, EB)], isrc_v)
        pltpu.sync_copy(dst_hbm.at[pl.ds(off, EB)], idst_v)
        pltpu.async_copy(f_hbm.at[isrc_v], rows_v, sem).wait()
        pltpu.sync_copy(rows_v, acc_sh.at[idst_v], add=True)
        return carry

    lax.fori_loop(0, NBLK, body, 0)
    plsc.subcore_barrier()
    pltpu.sync_copy(acc_sh.at[pl.ds(sid * ROWS_PER_TILE, ROWS_PER_TILE)],
                    out_hbm.at[cid, pl.ds(sid * ROWS_PER_TILE, ROWS_PER_TILE)])


@functools.cache
def _sc_prop():
    return pl.kernel(
        _sc_prop_body,
        mesh=_mesh(),
        out_type=jax.ShapeDtypeStruct((NC, NPAD, D), jnp.float32),
        scratch_types=[
            pltpu.VMEM((EB,), jnp.int32),
            pltpu.VMEM((EB,), jnp.int32),
            pltpu.VMEM((EB, D), jnp.float32),
            pltpu.VMEM_SHARED((NPAD, D), jnp.float32),
            pltpu.SemaphoreType.DMA,
        ],
    )


R = 512
GRID = (NPAD // R,)


def _dinv_block(degp_ref):
    d = degp_ref[0, :, 0:1] + degp_ref[1, :, 0:1] + 1.0
    return lax.rsqrt(d)


def _t1_body(degp_ref, x_ref, xp_ref):
    xp_ref[...] = _dinv_block(degp_ref) * x_ref[...]


_t1 = pl.pallas_call(
    _t1_body,
    grid=GRID,
    in_specs=[
        pl.BlockSpec((NC, R, DEGW), lambda i: (0, i, 0)),
        pl.BlockSpec((R, D), lambda i: (i, 0)),
    ],
    out_specs=pl.BlockSpec((R, D), lambda i: (i, 0)),
    out_shape=jax.ShapeDtypeStruct((N, D), jnp.float32),
)


def _t2_body(acc_ref, xp_ref, degp_ref, w_ref, b_ref, hp_ref):
    dinv = _dinv_block(degp_ref)
    s = dinv * (acc_ref[0] + acc_ref[1] + xp_ref[...])
    h = jnp.dot(s, w_ref[...], preferred_element_type=jnp.float32) + b_ref[...]
    hp_ref[...] = dinv * jnp.maximum(h, 0.0)


_t2 = pl.pallas_call(
    _t2_body,
    grid=GRID,
    in_specs=[
        pl.BlockSpec((NC, R, D), lambda i: (0, i, 0)),
        pl.BlockSpec((R, D), lambda i: (i, 0)),
        pl.BlockSpec((NC, R, DEGW), lambda i: (0, i, 0)),
        pl.BlockSpec((D, D), lambda i: (0, 0)),
        pl.BlockSpec((1, D), lambda i: (0, 0)),
    ],
    out_specs=pl.BlockSpec((R, D), lambda i: (i, 0)),
    out_shape=jax.ShapeDtypeStruct((N, D), jnp.float32),
)


def _t3_body(acc_ref, hp_ref, degp_ref, wmu_ref, bmu_ref, wls_ref, bls_ref,
             mu_ref, ls_ref):
    dinv = _dinv_block(degp_ref)
    g = dinv * (acc_ref[0] + acc_ref[1] + hp_ref[...])
    mu_ref[...] = jnp.dot(g, wmu_ref[...], preferred_element_type=jnp.float32) + bmu_ref[...]
    ls_ref[...] = jnp.dot(g, wls_ref[...], preferred_element_type=jnp.float32) + bls_ref[...]


_t3 = pl.pallas_call(
    _t3_body,
    grid=GRID,
    in_specs=[
        pl.BlockSpec((NC, R, D), lambda i: (0, i, 0)),
        pl.BlockSpec((R, D), lambda i: (i, 0)),
        pl.BlockSpec((NC, R, DEGW), lambda i: (0, i, 0)),
        pl.BlockSpec((D, DO), lambda i: (0, 0)),
        pl.BlockSpec((1, DO), lambda i: (0, 0)),
        pl.BlockSpec((D, DO), lambda i: (0, 0)),
        pl.BlockSpec((1, DO), lambda i: (0, 0)),
    ],
    out_specs=[
        pl.BlockSpec((R, DO), lambda i: (i, 0)),
        pl.BlockSpec((R, DO), lambda i: (i, 0)),
    ],
    out_shape=[
        jax.ShapeDtypeStruct((N, DO), jnp.float32),
        jax.ShapeDtypeStruct((N, DO), jnp.float32),
    ],
)


def kernel(x, edge_index, W1, b1, Wmu, bmu, Wls, bls):
    src = edge_index[0]
    dst = edge_index[1]
    degp = _sc_degree()(dst)
    xp = _t1(degp, x)
    prop = _sc_prop()
    acc1 = prop(xp, src, dst)
    hp = _t2(acc1, xp, degp, W1, b1.reshape(1, D))
    acc2 = prop(hp, src, dst)
    mu, logstd = _t3(acc2, hp, degp, Wmu, bmu.reshape(1, DO),
                     Wls, bls.reshape(1, DO))
    return (mu, logstd)


# speedup vs baseline: 32.1619x; 2.0892x over previous
import functools

import jax
import jax.numpy as jnp
from jax import lax
from jax.experimental import pallas as pl
from jax.experimental.pallas import tpu as pltpu
from jax.experimental.pallas import tpu_sc as plsc

N = 10000
E = 320000
D = 128
DO = 64

NC = 2
NS = 16
NW = NC * NS

NPAD = 10240
ROWS_PER_TILE = NPAD // NS
EB = 125
EPW = E // NW
NBLK = EPW // EB
HBLK = NBLK // 2
ZB = 80
DEGW = 16

def _mesh():
    return plsc.VectorSubcoreMesh(
        core_axis_name="c", subcore_axis_name="s", num_cores=NC, num_subcores=NS)


def _sc_degree_body(dst_hbm, out_hbm, idxs_v, ones_v, acc_sh, sem):
    cid = lax.axis_index("c")
    sid = lax.axis_index("s")
    wid = sid * NC + cid

    zero16 = jnp.zeros((16,), jnp.float32)
    for i in range(EB):
        ones_v[i, :] = zero16
    zslice = ones_v.at[pl.ds(0, ZB)]
    for k in range(ROWS_PER_TILE // ZB):
        pltpu.sync_copy(zslice, acc_sh.at[pl.ds(sid * ROWS_PER_TILE + k * ZB, ZB)])
    one16 = jnp.ones((16,), jnp.float32)
    for i in range(EB):
        ones_v[i, :] = one16
    pltpu.sync_copy(dst_hbm.at[wid], idxs_v)
    plsc.subcore_barrier()

    FIRE = 8

    def body(r, carry):
        for k in range(FIRE):
            pltpu.async_copy(ones_v, acc_sh.at[idxs_v.at[r * FIRE + k]], sem,
                             add=True)
        for k in range(FIRE):
            pltpu.make_async_copy(ones_v, acc_sh.at[idxs_v.at[r * FIRE + k]],
                                  sem).wait()
        return carry

    lax.fori_loop(0, NBLK // FIRE, body, 0)
    plsc.subcore_barrier()
    pltpu.sync_copy(acc_sh.at[pl.ds(sid * ROWS_PER_TILE, ROWS_PER_TILE)],
                    out_hbm.at[cid, pl.ds(sid * ROWS_PER_TILE, ROWS_PER_TILE)])


@functools.cache
def _sc_degree():
    return pl.kernel(
        _sc_degree_body,
        mesh=_mesh(),
        out_type=jax.ShapeDtypeStruct((NC, NPAD, DEGW), jnp.float32),
        scratch_types=[
            pltpu.VMEM((NBLK, EB), jnp.int32),
            pltpu.VMEM((EB, DEGW), jnp.float32),
            pltpu.VMEM_SHARED((NPAD, DEGW), jnp.float32),
            pltpu.SemaphoreType.DMA,
        ],
    )


def _sc_prop_body(f_hbm, src_hbm, dst_hbm, out_hbm, srcs_v, dsts_v,
                  rows0_v, rows1_v, acc_sh, gsem0, gsem1):
    cid = lax.axis_index("c")
    sid = lax.axis_index("s")
    wid = sid * NC + cid

    zero16 = jnp.zeros((16,), jnp.float32)
    for i in range(ZB):
        for j in range(D // 16):
            rows0_v[i, pl.ds(j * 16, 16)] = zero16
    zslice = rows0_v.at[pl.ds(0, ZB)]
    for k in range(ROWS_PER_TILE // ZB):
        pltpu.sync_copy(zslice, acc_sh.at[pl.ds(sid * ROWS_PER_TILE + k * ZB, ZB)])
    plsc.subcore_barrier()

    for h in range(2):
        pltpu.sync_copy(src_hbm.at[wid, pl.ds(h * HBLK, HBLK)], srcs_v)
        pltpu.sync_copy(dst_hbm.at[wid, pl.ds(h * HBLK, HBLK)], dsts_v)
        pltpu.async_copy(f_hbm.at[srcs_v.at[0]], rows0_v, gsem0)

        def rnd(r, carry):
            j0 = 2 * r
            pltpu.make_async_copy(f_hbm.at[srcs_v.at[j0]], rows0_v, gsem0).wait()
            pltpu.async_copy(f_hbm.at[srcs_v.at[j0 + 1]], rows1_v, gsem1)
            pltpu.sync_copy(rows0_v, acc_sh.at[dsts_v.at[j0]], add=True)
            pltpu.make_async_copy(f_hbm.at[srcs_v.at[j0 + 1]], rows1_v, gsem1).wait()

            @pl.when(r < HBLK // 2 - 1)
            def _():
                pltpu.async_copy(f_hbm.at[srcs_v.at[j0 + 2]], rows0_v, gsem0)

            pltpu.sync_copy(rows1_v, acc_sh.at[dsts_v.at[j0 + 1]], add=True)
            return carry

        lax.fori_loop(0, HBLK // 2, rnd, 0)
    plsc.subcore_barrier()
    pltpu.sync_copy(acc_sh.at[pl.ds(sid * ROWS_PER_TILE, ROWS_PER_TILE)],
                    out_hbm.at[cid, pl.ds(sid * ROWS_PER_TILE, ROWS_PER_TILE)])


@functools.cache
def _sc_prop():
    return pl.kernel(
        _sc_prop_body,
        mesh=_mesh(),
        out_type=jax.ShapeDtypeStruct((NC, NPAD, D), jnp.float32),
        scratch_types=[
            pltpu.VMEM((HBLK, EB), jnp.int32),
            pltpu.VMEM((HBLK, EB), jnp.int32),
            pltpu.VMEM((EB, D), jnp.float32),
            pltpu.VMEM((EB, D), jnp.float32),
            pltpu.VMEM_SHARED((NPAD, D), jnp.float32),
            pltpu.SemaphoreType.DMA,
            pltpu.SemaphoreType.DMA,
        ],
    )


R = 512
GRID = (NPAD // R,)


def _dinv_block(degp_ref):
    d = degp_ref[0, :, 0:1] + degp_ref[1, :, 0:1] + 1.0
    return lax.rsqrt(d)


def _t1_body(degp_ref, x_ref, xp_ref):
    xp_ref[...] = _dinv_block(degp_ref) * x_ref[...]


_t1 = pl.pallas_call(
    _t1_body,
    grid=GRID,
    in_specs=[
        pl.BlockSpec((NC, R, DEGW), lambda i: (0, i, 0)),
        pl.BlockSpec((R, D), lambda i: (i, 0)),
    ],
    out_specs=pl.BlockSpec((R, D), lambda i: (i, 0)),
    out_shape=jax.ShapeDtypeStruct((N, D), jnp.float32),
)


def _t2_body(acc_ref, xp_ref, degp_ref, w_ref, b_ref, hp_ref):
    dinv = _dinv_block(degp_ref)
    s = dinv * (acc_ref[0] + acc_ref[1] + xp_ref[...])
    h = jnp.dot(s, w_ref[...], preferred_element_type=jnp.float32) + b_ref[...]
    hp_ref[...] = dinv * jnp.maximum(h, 0.0)


_t2 = pl.pallas_call(
    _t2_body,
    grid=GRID,
    in_specs=[
        pl.BlockSpec((NC, R, D), lambda i: (0, i, 0)),
        pl.BlockSpec((R, D), lambda i: (i, 0)),
        pl.BlockSpec((NC, R, DEGW), lambda i: (0, i, 0)),
        pl.BlockSpec((D, D), lambda i: (0, 0)),
        pl.BlockSpec((1, D), lambda i: (0, 0)),
    ],
    out_specs=pl.BlockSpec((R, D), lambda i: (i, 0)),
    out_shape=jax.ShapeDtypeStruct((N, D), jnp.float32),
)


def _t3_body(acc_ref, hp_ref, degp_ref, wmu_ref, bmu_ref, wls_ref, bls_ref,
             mu_ref, ls_ref):
    dinv = _dinv_block(degp_ref)
    g = dinv * (acc_ref[0] + acc_ref[1] + hp_ref[...])
    mu_ref[...] = jnp.dot(g, wmu_ref[...], preferred_element_type=jnp.float32) + bmu_ref[...]
    ls_ref[...] = jnp.dot(g, wls_ref[...], preferred_element_type=jnp.float32) + bls_ref[...]


_t3 = pl.pallas_call(
    _t3_body,
    grid=GRID,
    in_specs=[
        pl.BlockSpec((NC, R, D), lambda i: (0, i, 0)),
        pl.BlockSpec((R, D), lambda i: (i, 0)),
        pl.BlockSpec((NC, R, DEGW), lambda i: (0, i, 0)),
        pl.BlockSpec((D, DO), lambda i: (0, 0)),
        pl.BlockSpec((1, DO), lambda i: (0, 0)),
        pl.BlockSpec((D, DO), lambda i: (0, 0)),
        pl.BlockSpec((1, DO), lambda i: (0, 0)),
    ],
    out_specs=[
        pl.BlockSpec((R, DO), lambda i: (i, 0)),
        pl.BlockSpec((R, DO), lambda i: (i, 0)),
    ],
    out_shape=[
        jax.ShapeDtypeStruct((N, DO), jnp.float32),
        jax.ShapeDtypeStruct((N, DO), jnp.float32),
    ],
)


def kernel(x, edge_index, W1, b1, Wmu, bmu, Wls, bls):
    src = edge_index[0].reshape(NW, NBLK, EB)
    dst = edge_index[1].reshape(NW, NBLK, EB)
    degp = _sc_degree()(dst)
    xp = _t1(degp, x)
    prop = _sc_prop()
    acc1 = prop(xp, src, dst)
    hp = _t2(acc1, xp, degp, W1, b1.reshape(1, D))
    acc2 = prop(hp, src, dst)
    mu, logstd = _t3(acc2, hp, degp, Wmu, bmu.reshape(1, DO),
                     Wls, bls.reshape(1, DO))
    return (mu, logstd)
